# baseline (speedup 1.0000x reference)
import functools

import jax
import jax.numpy as jnp
from jax import lax
from jax.experimental import pallas as pl
from jax.experimental.pallas import tpu as pltpu
from jax.experimental.pallas import tpu_sc as plsc

_N = 10000
_E = 320000
_D = 128
_G = 64
_EPS = 1e-5

_NC = 2
_NS = 16
_NW = _NC * _NS
_K = 64
_SUP = 8
_NSUP = 20
_NCHUNK = _NSUP * _SUP
_EPWP = _NCHUNK * _K
_EPW = _E // _NW
_NB = 4
_RPT = 624
_ZR = 48


def _segsum_body(x_hbm, src_hbm, dst_hbm, out_hbm, acc, zbuf, src_v, dst_v,
                 rows_v, semi, semg, sems):
  cid = lax.axis_index("c")
  sid = lax.axis_index("s")
  wid = sid * _NC + cid

  def issue_idx(s, slot):
    pltpu.async_copy(src_hbm.at[wid].at[pl.ds(s * _SUP, _SUP)],
                     src_v.at[slot], semi[slot])
    pltpu.async_copy(dst_hbm.at[wid].at[pl.ds(s * _SUP, _SUP)],
                     dst_v.at[slot], semi[slot])

  def wait_idx(slot):
    pltpu.make_async_copy(src_hbm.at[wid].at[pl.ds(0, _SUP)],
                          src_v.at[slot], semi[slot]).wait()
    pltpu.make_async_copy(dst_hbm.at[wid].at[pl.ds(0, _SUP)],
                          dst_v.at[slot], semi[slot]).wait()

  def issue_gather(slot, b, p):
    pltpu.async_copy(x_hbm.at[src_v.at[slot].at[b]], rows_v.at[p], semg[p])

  def wait_gather(p):
    pltpu.make_async_copy(x_hbm.at[src_v.at[0].at[0]], rows_v.at[p],
                          semg[p]).wait()

  def issue_scatter(slot, b, p):
    pltpu.async_copy(rows_v.at[p], acc.at[dst_v.at[slot].at[b]], sems[p],
                     add=True)

  def wait_scatter(p):
    pltpu.make_async_copy(rows_v.at[p], acc.at[dst_v.at[0].at[0]],
                          sems[p]).wait()

  issue_idx(0, 0)
  issue_idx(1, 1)

  @pl.loop(0, _ZR)
  def _(i):
    @pl.loop(0, _D // 16)
    def _(j):
      zbuf[i, pl.ds(j * 16, 16)] = jnp.zeros((16,), jnp.float32)

  @pl.loop(0, _RPT // _ZR)
  def _(i):
    off = pl.multiple_of(sid * _RPT + i * _ZR, 8)
    pltpu.sync_copy(zbuf, acc.at[pl.ds(off, _ZR)])

  @pl.when(sid == _NS - 1)
  def _():
    pltpu.sync_copy(zbuf.at[pl.ds(0, _N - _NS * _RPT)],
                    acc.at[pl.ds(_NS * _RPT, _N - _NS * _RPT)])

  wait_idx(0)
  issue_gather(0, 0, 0)

  plsc.subcore_barrier()

  @pl.loop(0, _NSUP // 2)
  def _(it):
    for sp in range(2):
      s = it * 2 + sp
      for b in range(_SUP):
        i = s * _SUP + b
        if b == _SUP - 1:
          @pl.when(s + 1 < _NSUP)
          def _():
            wait_idx((sp + 1) % 2)

        @pl.when(i >= _NB - 1)
        def _():
          wait_scatter((b + 1) % _NB)

        @pl.when(i + 1 < _NCHUNK)
        def _():
          if b == _SUP - 1:
            issue_gather((sp + 1) % 2, 0, (b + 1) % _NB)
          else:
            issue_gather(sp, b + 1, (b + 1) % _NB)

        wait_gather(b % _NB)
        issue_scatter(sp, b, b % _NB)

      @pl.when(s + 2 < _NSUP)
      def _():
        issue_idx(s + 2, sp)

  for j in range(_NB - 1):
    wait_scatter((_NCHUNK - (_NB - 1) + j) % _NB)

  plsc.subcore_barrier()

  @pl.loop(0, _RPT // _ZR)
  def _(i):
    off = pl.multiple_of(sid * _RPT + i * _ZR, 8)
    pltpu.sync_copy(acc.at[pl.ds(off, _ZR)],
                    out_hbm.at[pl.ds(cid * _N + off, _ZR)])

  @pl.when(sid == _NS - 1)
  def _():
    pltpu.sync_copy(acc.at[pl.ds(_NS * _RPT, _N - _NS * _RPT)],
                    out_hbm.at[pl.ds(cid * _N + _NS * _RPT, _N - _NS * _RPT)])


@functools.cache
def _get_segsum():
  return pl.kernel(
      _segsum_body,
      out_type=jax.ShapeDtypeStruct((_NC * _N, _D), jnp.float32),
      mesh=plsc.VectorSubcoreMesh(core_axis_name="c", subcore_axis_name="s",
                                  num_cores=_NC, num_subcores=_NS),
      scratch_types=[
          pltpu.VMEM_SHARED((_N + _K, _D), jnp.float32),
          pltpu.VMEM((_ZR, _D), jnp.float32),
          pltpu.VMEM((2, _SUP, _K), jnp.int32),
          pltpu.VMEM((2, _SUP, _K), jnp.int32),
          pltpu.VMEM((_NB, _K, _D), jnp.float32),
          [pltpu.SemaphoreType.DMA] * 2,
          [pltpu.SemaphoreType.DMA] * _NB,
          [pltpu.SemaphoreType.DMA] * _NB,
      ],
  )


def _edge_tables(edge_index):
  src = edge_index[0].reshape(_NW, _EPW)
  dst = edge_index[1].reshape(_NW, _EPW)
  npad = _EPWP - _EPW
  pad_dst = _N + (jnp.arange(npad, dtype=jnp.int32) % _K)
  src_t = jnp.concatenate(
      [src, jnp.full((_NW, npad), _N, jnp.int32)], axis=1)
  dst_t = jnp.concatenate(
      [dst, jnp.broadcast_to(pad_dst, (_NW, npad))], axis=1)
  return (src_t.reshape(_NW, _NCHUNK, _K), dst_t.reshape(_NW, _NCHUNK, _K))


def _padx(x):
  return jnp.concatenate([x, jnp.zeros((8, _D), jnp.float32)], axis=0)


def _dense_body(x_ref, p_ref, Wa_ref, ba_ref, Wb_ref, bb_ref, g_ref, be_ref,
                o_ref):
  h0 = x_ref[...] + p_ref[0:_N, :] + p_ref[_N:2 * _N, :]
  t = jnp.dot(h0, Wa_ref[...], preferred_element_type=jnp.float32)
  t = jnp.maximum(t + ba_ref[...], 0.0)
  h = jnp.dot(t, Wb_ref[...], preferred_element_type=jnp.float32) + bb_ref[...]
  m = jnp.mean(h, axis=0, keepdims=True)
  c = h - m
  v = jnp.mean(c * c, axis=0, keepdims=True)
  hn = c * lax.rsqrt(v + _EPS) * g_ref[...] + be_ref[...]
  o_ref[...] = jnp.maximum(hn, 0.0)


_dense = pl.pallas_call(
    _dense_body,
    out_shape=jax.ShapeDtypeStruct((_N, _D), jnp.float32),
)


def _pool_body(h_ref, batch_ref, Wf_ref, bf_ref, o_ref):
  gids = lax.broadcasted_iota(jnp.int32, (_G, _N), 0)
  mask = (gids == batch_ref[...]).astype(jnp.float32)
  sums = jnp.dot(mask, h_ref[...], preferred_element_type=jnp.float32)
  counts = jnp.sum(mask, axis=1, keepdims=True)
  pooled = sums / jnp.maximum(counts, 1.0)
  o_ref[...] = (
      jnp.dot(pooled, Wf_ref[...], preferred_element_type=jnp.float32)
      + bf_ref[...])


_pool = pl.pallas_call(
    _pool_body,
    out_shape=jax.ShapeDtypeStruct((_G, _D), jnp.float32),
)


@jax.jit
def kernel(x, edge_index, batch, W1, b1, W2, b2, g1, be1, W3, b3, W4, b4, g2,
           be2, Wf, bf):
  src_t, dst_t = _edge_tables(edge_index)
  _segsum = _get_segsum()
  p1 = _segsum(_padx(x), src_t, dst_t)
  h1 = _dense(x, p1, W1, b1.reshape(1, _D), W2, b2.reshape(1, _D),
              g1.reshape(1, _D), be1.reshape(1, _D))
  p2 = _segsum(_padx(h1), src_t, dst_t)
  h2 = _dense(h1, p2, W3, b3.reshape(1, _D), W4, b4.reshape(1, _D),
              g2.reshape(1, _D), be2.reshape(1, _D))
  return _pool(h2, batch.reshape(1, _N), Wf, bf.reshape(1, _D))


# speedup vs baseline: 2.5354x; 2.5354x over previous
import functools

import jax
import jax.numpy as jnp
from jax import lax
from jax.experimental import pallas as pl
from jax.experimental.pallas import tpu as pltpu
from jax.experimental.pallas import tpu_sc as plsc

_N = 10000
_E = 320000
_D = 128
_G = 64
_EPS = 1e-5

_NC = 2
_NS = 16
_NW = _NC * _NS
_EPW = _E // _NW
_K = 80
_NCHUNK = _EPW // _K
_NB = 4
_RPT = 624
_ZR = 48


def _segsum_body(x_hbm, src_hbm, dst_hbm, out_hbm, acc, zbuf, src_v, dst_v,
                 rows_v, semg, sems):
  cid = lax.axis_index("c")
  sid = lax.axis_index("s")
  wid = sid * _NC + cid
  base = wid * _EPW

  def load_idx(i, q):
    off = pl.multiple_of(base + i * _K, 8)
    pltpu.sync_copy(src_hbm.at[pl.ds(off, _K)], src_v[q])
    pltpu.sync_copy(dst_hbm.at[pl.ds(off, _K)], dst_v[q])

  def issue_gather(q):
    pltpu.async_copy(x_hbm.at[src_v[q]], rows_v.at[q], semg[q])

  def wait_gather(q):
    pltpu.make_async_copy(x_hbm.at[src_v[q]], rows_v.at[q], semg[q]).wait()

  def issue_scatter(q):
    pltpu.async_copy(rows_v.at[q], acc.at[dst_v[q]], sems[q], add=True)

  def wait_scatter(q):
    pltpu.make_async_copy(rows_v.at[q], acc.at[dst_v[q]], sems[q]).wait()

  @pl.loop(0, _ZR)
  def _(i):
    @pl.loop(0, _D // 16)
    def _(j):
      zbuf[i, pl.ds(j * 16, 16)] = jnp.zeros((16,), jnp.float32)

  @pl.loop(0, _RPT // _ZR)
  def _(i):
    off = pl.multiple_of(sid * _RPT + i * _ZR, 8)
    pltpu.sync_copy(zbuf, acc.at[pl.ds(off, _ZR)])

  @pl.when(sid == _NS - 1)
  def _():
    pltpu.sync_copy(zbuf.at[pl.ds(0, _N - _NS * _RPT)],
                    acc.at[pl.ds(_NS * _RPT, _N - _NS * _RPT)])

  load_idx(0, 0)
  issue_gather(0)

  plsc.subcore_barrier()

  @pl.loop(0, _NCHUNK // _NB)
  def _(it):
    for b in range(_NB):
      i = it * _NB + b
      q = (b + 1) % _NB

      @pl.when(i >= _NB - 1)
      def _():
        wait_scatter(q)

      load_idx(i + 1, q)
      issue_gather(q)
      wait_gather(b)
      issue_scatter(b)

  lastq = (_NCHUNK - 1) % _NB
  wait_gather(lastq)
  issue_scatter(lastq)
  for j in range(_NB):
    wait_scatter((_NCHUNK + j) % _NB)

  plsc.subcore_barrier()

  @pl.loop(0, _RPT // _ZR)
  def _(i):
    off = pl.multiple_of(sid * _RPT + i * _ZR, 8)
    pltpu.sync_copy(acc.at[pl.ds(off, _ZR)],
                    out_hbm.at[pl.ds(cid * _N + off, _ZR)])

  @pl.when(sid == _NS - 1)
  def _():
    pltpu.sync_copy(acc.at[pl.ds(_NS * _RPT, _N - _NS * _RPT)],
                    out_hbm.at[pl.ds(cid * _N + _NS * _RPT, _N - _NS * _RPT)])


@functools.cache
def _get_segsum():
  return pl.kernel(
      _segsum_body,
      out_type=jax.ShapeDtypeStruct((_NC * _N, _D), jnp.float32),
      mesh=plsc.VectorSubcoreMesh(core_axis_name="c", subcore_axis_name="s",
                                  num_cores=_NC, num_subcores=_NS),
      scratch_types=[
          pltpu.VMEM_SHARED((_N, _D), jnp.float32),
          pltpu.VMEM((_ZR, _D), jnp.float32),
          [pltpu.VMEM((_K,), jnp.int32)] * _NB,
          [pltpu.VMEM((_K,), jnp.int32)] * _NB,
          pltpu.VMEM((_NB, _K, _D), jnp.float32),
          [pltpu.SemaphoreType.DMA] * _NB,
          [pltpu.SemaphoreType.DMA] * _NB,
      ],
  )


def _dense_body(x_ref, p_ref, Wa_ref, ba_ref, Wb_ref, bb_ref, g_ref, be_ref,
                o_ref):
  h0 = x_ref[...] + p_ref[0:_N, :] + p_ref[_N:2 * _N, :]
  t = jnp.dot(h0, Wa_ref[...], preferred_element_type=jnp.float32)
  t = jnp.maximum(t + ba_ref[...], 0.0)
  h = jnp.dot(t, Wb_ref[...], preferred_element_type=jnp.float32) + bb_ref[...]
  m = jnp.mean(h, axis=0, keepdims=True)
  c = h - m
  v = jnp.mean(c * c, axis=0, keepdims=True)
  hn = c * lax.rsqrt(v + _EPS) * g_ref[...] + be_ref[...]
  o_ref[...] = jnp.maximum(hn, 0.0)


_dense = pl.pallas_call(
    _dense_body,
    out_shape=jax.ShapeDtypeStruct((_N, _D), jnp.float32),
)


def _pool_body(h_ref, batch_ref, Wf_ref, bf_ref, o_ref):
  gids = lax.broadcasted_iota(jnp.int32, (_G, _N), 0)
  mask = (gids == batch_ref[...]).astype(jnp.float32)
  sums = jnp.dot(mask, h_ref[...], preferred_element_type=jnp.float32)
  counts = jnp.sum(mask, axis=1, keepdims=True)
  pooled = sums / jnp.maximum(counts, 1.0)
  o_ref[...] = (
      jnp.dot(pooled, Wf_ref[...], preferred_element_type=jnp.float32)
      + bf_ref[...])


_pool = pl.pallas_call(
    _pool_body,
    out_shape=jax.ShapeDtypeStruct((_G, _D), jnp.float32),
)


@jax.jit
def kernel(x, edge_index, batch, W1, b1, W2, b2, g1, be1, W3, b3, W4, b4, g2,
           be2, Wf, bf):
  src = edge_index[0]
  dst = edge_index[1]
  _segsum = _get_segsum()
  p1 = _segsum(x, src, dst)
  h1 = _dense(x, p1, W1, b1.reshape(1, _D), W2, b2.reshape(1, _D),
              g1.reshape(1, _D), be1.reshape(1, _D))
  p2 = _segsum(h1, src, dst)
  h2 = _dense(h1, p2, W3, b3.reshape(1, _D), W4, b4.reshape(1, _D),
              g2.reshape(1, _D), be2.reshape(1, _D))
  return _pool(h2, batch.reshape(1, _N), Wf, bf.reshape(1, _D))
